# baseline (speedup 1.0000x reference)
import functools

import jax
import jax.numpy as jnp
from jax import lax
from jax.experimental import pallas as pl
from jax.experimental.pallas import tpu as pltpu
from jax.experimental.pallas import tpu_sc as plsc


def _stats_body(x_ref, wt1_ref, wt2_ref, wc1t_ref, wc2t_ref, ta_ref, ca_ref):
    xb = x_ref[0]
    f, h, w, c = xb.shape
    hw = h * w
    s = jnp.sum(xb, axis=(1, 2))
    mx = jnp.max(xb, axis=(1, 2))

    avg_t = jnp.sum(s, axis=1, keepdims=True) * (1.0 / (c * hw))
    max_t = jnp.max(mx, axis=1, keepdims=True)
    vt = jnp.concatenate([avg_t, max_t], axis=1)
    ht = jnp.maximum(jnp.dot(wt1_ref[...], vt,
                             preferred_element_type=jnp.float32), 0.0)
    ot = jnp.dot(wt2_ref[...], ht, preferred_element_type=jnp.float32)
    ta = jax.nn.sigmoid(ot[:, 0:1] + ot[:, 1:2])

    avg_c = jnp.sum(ta * s, axis=0, keepdims=True) * (1.0 / (f * hw))
    max_c = jnp.max(ta * mx, axis=0, keepdims=True)
    vc = jnp.concatenate([avg_c, max_c], axis=0)
    hc = jnp.maximum(jnp.dot(vc, wc1t_ref[...],
                             preferred_element_type=jnp.float32), 0.0)
    oc = jnp.dot(hc, wc2t_ref[...], preferred_element_type=jnp.float32)
    ca = jax.nn.sigmoid(oc[0:1, :] + oc[1:2, :])

    ta_ref[0] = jnp.transpose(ta)
    ca_ref[0] = ca


def _sc_wta_body(ta_hbm, ca_hbm, tag_hbm, cag_hbm, tav, cav, tagv, cagv):
    info = plsc.get_sparse_core_info()
    nc = info.num_cores
    wid = lax.axis_index("s") * nc + lax.axis_index("c")
    nb = ta_hbm.shape[0]

    @pl.when(wid < nb)
    def _():
        pltpu.sync_copy(ta_hbm.at[wid], tav)
        pltpu.sync_copy(ca_hbm.at[wid], cav)

        lane = jnp.arange(16, dtype=jnp.int32)

        zero = lane * 0
        one = zero + 1

        ta = tav[...]
        rank = zero
        for j in range(16):
            bj = ta.at[zero + j].get(mode="promise_in_bounds")
            beats = (bj > ta) | ((bj == ta) & (j < lane))
            rank = rank + jnp.where(beats, one, zero)
        tagv[...] = jnp.where(rank < 14, ta, ta * 0.0)

        chunks = [cav[pl.ds(16 * t, 16)] for t in range(8)]
        gidx = [lane + 16 * t for t in range(8)]
        ranks = [zero for _ in range(8)]
        for j in range(128):
            cj, lj = divmod(j, 16)
            bj = chunks[cj].at[zero + lj].get(mode="promise_in_bounds")
            for t in range(8):
                beats = (bj > chunks[t]) | ((bj == chunks[t]) & (j < gidx[t]))
                ranks[t] = ranks[t] + jnp.where(beats, one, zero)
        for t in range(8):
            cagv[pl.ds(16 * t, 16)] = jnp.where(
                ranks[t] < 102, chunks[t], chunks[t] * 0.0)

        pltpu.sync_copy(tagv, tag_hbm.at[wid])
        pltpu.sync_copy(cagv, cag_hbm.at[wid])


def _scale_body(x_ref, tag_ref, cag_ref, o_ref):
    gt_col = jnp.transpose(tag_ref[0])
    gc_row = cag_ref[0]
    g = (gt_col * gc_row)[:, None, None, :]
    o_ref[0] = x_ref[0] * g


def kernel(x, w_ta1, w_ta2, w_ca1, w_ca2):
    b, f, c, h, w = x.shape
    xt = jnp.transpose(x, (0, 1, 3, 4, 2))

    ta3, ca3 = pl.pallas_call(
        _stats_body,
        grid=(b,),
        in_specs=[
            pl.BlockSpec((1, f, h, w, c), lambda i: (i, 0, 0, 0, 0)),
            pl.BlockSpec((f, f), lambda i: (0, 0)),
            pl.BlockSpec((f, f), lambda i: (0, 0)),
            pl.BlockSpec((c, c), lambda i: (0, 0)),
            pl.BlockSpec((c, c), lambda i: (0, 0)),
        ],
        out_specs=[
            pl.BlockSpec((1, 1, f), lambda i: (i, 0, 0)),
            pl.BlockSpec((1, 1, c), lambda i: (i, 0, 0)),
        ],
        out_shape=[
            jax.ShapeDtypeStruct((b, 1, f), x.dtype),
            jax.ShapeDtypeStruct((b, 1, c), x.dtype),
        ],
    )(xt, w_ta1, w_ta2, w_ca1.T, w_ca2.T)

    mesh = plsc.VectorSubcoreMesh(core_axis_name="c", subcore_axis_name="s")
    sc_gate = functools.partial(
        pl.kernel,
        mesh=mesh,
        out_type=[
            jax.ShapeDtypeStruct((b, f), jnp.float32),
            jax.ShapeDtypeStruct((b, c), jnp.float32),
        ],
        scratch_types=[
            pltpu.VMEM((f,), jnp.float32),
            pltpu.VMEM((c,), jnp.float32),
            pltpu.VMEM((f,), jnp.float32),
            pltpu.VMEM((c,), jnp.float32),
        ],
    )(_sc_wta_body)
    tag, cag = sc_gate(ta3.reshape(b, f), ca3.reshape(b, c))

    out_t = pl.pallas_call(
        _scale_body,
        grid=(b,),
        in_specs=[
            pl.BlockSpec((1, f, h, w, c), lambda i: (i, 0, 0, 0, 0)),
            pl.BlockSpec((1, 1, f), lambda i: (i, 0, 0)),
            pl.BlockSpec((1, 1, c), lambda i: (i, 0, 0)),
        ],
        out_specs=pl.BlockSpec((1, f, h, w, c), lambda i: (i, 0, 0, 0, 0)),
        out_shape=jax.ShapeDtypeStruct((b, f, h, w, c), x.dtype),
    )(xt, tag.reshape(b, 1, f), cag.reshape(b, 1, c))
    return jnp.transpose(out_t, (0, 1, 4, 2, 3))


# speedup vs baseline: 1.8202x; 1.8202x over previous
import jax
import jax.numpy as jnp
from jax import lax
from jax.experimental import pallas as pl
from jax.experimental.pallas import tpu as pltpu


def _wta_gate(v_col):
    n = v_col.shape[0]
    ratio = 0.9 if n == 16 else 0.8
    k = int(n * ratio)
    a = jnp.broadcast_to(v_col, (n, n))
    b = jnp.transpose(a)
    row = lax.broadcasted_iota(jnp.int32, (n, n), 0)
    col = lax.broadcasted_iota(jnp.int32, (n, n), 1)
    beats = (b > a) | ((b == a) & (col < row))
    rank = jnp.sum(beats.astype(jnp.float32), axis=1, keepdims=True)
    mask = jnp.where(rank < float(k), 1.0, 0.0)
    return v_col * mask


_NQ = 4


def _rm_sew_body(x_hbm, wt1_ref, wt2_ref, wc1t_ref, wc2t_ref, o_hbm,
                 xbuf, obuf, in_sem, out_sem):
    nb, f, h, w, c = x_hbm.shape
    hw = h * w
    fq = f // _NQ
    i = pl.program_id(0)
    p = lax.rem(i, 2)
    pn = lax.rem(i + 1, 2)

    def in_copy(src_b, dst_p, q):
        return pltpu.make_async_copy(
            x_hbm.at[src_b, pl.ds(q * fq, fq)],
            xbuf.at[dst_p, pl.ds(q * fq, fq)],
            in_sem.at[dst_p, q])

    def out_copy(dst_b, q, qp):
        return pltpu.make_async_copy(
            obuf.at[qp],
            o_hbm.at[dst_b, pl.ds(q * fq, fq)],
            out_sem.at[qp])

    @pl.when(i == 0)
    def _prologue():
        for q in range(_NQ):
            in_copy(0, 0, q).start()

    s_parts, mx_parts = [], []
    for q in range(_NQ):
        in_copy(i, p, q).wait()
        xq = xbuf[p, q * fq:(q + 1) * fq]
        s_parts.append(jnp.sum(xq, axis=(1, 2)))
        mx_parts.append(jnp.max(xq, axis=(1, 2)))

        @pl.when(i < nb - 1)
        def _prefetch():
            in_copy(i + 1, pn, q).start()

    s = jnp.concatenate(s_parts, axis=0)
    mx = jnp.concatenate(mx_parts, axis=0)

    avg_t = jnp.sum(s, axis=1, keepdims=True) * (1.0 / (c * hw))
    max_t = jnp.max(mx, axis=1, keepdims=True)
    vt = jnp.concatenate([avg_t, max_t], axis=1)
    ht = jnp.maximum(jnp.dot(wt1_ref[...], vt,
                             preferred_element_type=jnp.float32), 0.0)
    ot = jnp.dot(wt2_ref[...], ht, preferred_element_type=jnp.float32)
    ta = jax.nn.sigmoid(ot[:, 0:1] + ot[:, 1:2])

    avg_c = jnp.sum(ta * s, axis=0, keepdims=True) * (1.0 / (f * hw))
    max_c = jnp.max(ta * mx, axis=0, keepdims=True)
    vc = jnp.concatenate([avg_c, max_c], axis=0)
    hc = jnp.maximum(jnp.dot(vc, wc1t_ref[...],
                             preferred_element_type=jnp.float32), 0.0)
    oc = jnp.dot(hc, wc2t_ref[...], preferred_element_type=jnp.float32)
    ca = jax.nn.sigmoid(oc[0:1, :] + oc[1:2, :])

    g_t = _wta_gate(ta)
    g_c = jnp.transpose(_wta_gate(jnp.transpose(ca)))
    g = (g_t * g_c)[:, None, None, :]

    for q in range(_NQ):
        qp = q % 2
        if q >= 2:
            out_copy(i, q - 2, qp).wait()
        else:
            @pl.when(i > 0)
            def _drain():
                out_copy(i - 1, q + 2, qp).wait()
        obuf[qp] = xbuf[p, q * fq:(q + 1) * fq] * g[q * fq:(q + 1) * fq]
        out_copy(i, q, qp).start()

    @pl.when(i == nb - 1)
    def _epilogue():
        out_copy(i, _NQ - 2, 0).wait()
        out_copy(i, _NQ - 1, 1).wait()


def kernel(x, w_ta1, w_ta2, w_ca1, w_ca2):
    b, f, c, h, w = x.shape
    fq = f // _NQ
    xt = jnp.transpose(x, (0, 1, 3, 4, 2))
    out_t = pl.pallas_call(
        _rm_sew_body,
        grid=(b,),
        in_specs=[
            pl.BlockSpec(memory_space=pl.ANY),
            pl.BlockSpec((f, f), lambda i: (0, 0)),
            pl.BlockSpec((f, f), lambda i: (0, 0)),
            pl.BlockSpec((c, c), lambda i: (0, 0)),
            pl.BlockSpec((c, c), lambda i: (0, 0)),
        ],
        out_specs=pl.BlockSpec(memory_space=pl.ANY),
        out_shape=jax.ShapeDtypeStruct((b, f, h, w, c), x.dtype),
        scratch_shapes=[
            pltpu.VMEM((2, f, h, w, c), jnp.float32),
            pltpu.VMEM((2, fq, h, w, c), jnp.float32),
            pltpu.SemaphoreType.DMA((2, _NQ)),
            pltpu.SemaphoreType.DMA((2,)),
        ],
    )(xt, w_ta1, w_ta2, w_ca1.T, w_ca2.T)
    return jnp.transpose(out_t, (0, 1, 4, 2, 3))


# speedup vs baseline: 1.8858x; 1.0361x over previous
import jax
import jax.numpy as jnp
from jax import lax
from jax.experimental import pallas as pl


def _wta_gate(v_col):
    n = v_col.shape[0]
    ratio = 0.9 if n == 16 else 0.8
    k = int(n * ratio)
    a = jnp.broadcast_to(v_col, (n, n))
    b = jnp.transpose(a)
    row = lax.broadcasted_iota(jnp.int32, (n, n), 0)
    col = lax.broadcasted_iota(jnp.int32, (n, n), 1)
    beats = (b > a) | ((b == a) & (col < row))
    rank = jnp.sum(beats.astype(jnp.float32), axis=1, keepdims=True)
    mask = jnp.where(rank < float(k), 1.0, 0.0)
    return v_col * mask


def _rm_sew_body(x_ref, wt1_ref, wt2_ref, wc1t_ref, wc2t_ref, o_ref):
    xb = x_ref[0]
    f, h, w, c = xb.shape
    hw = h * w
    s = jnp.sum(xb, axis=(1, 2))
    mx = jnp.max(xb, axis=(1, 2))

    avg_t = jnp.sum(s, axis=1, keepdims=True) * (1.0 / (c * hw))
    max_t = jnp.max(mx, axis=1, keepdims=True)
    vt = jnp.concatenate([avg_t, max_t], axis=1)
    ht = jnp.maximum(jnp.dot(wt1_ref[...], vt,
                             preferred_element_type=jnp.float32), 0.0)
    ot = jnp.dot(wt2_ref[...], ht, preferred_element_type=jnp.float32)
    ta = jax.nn.sigmoid(ot[:, 0:1] + ot[:, 1:2])

    avg_c = jnp.sum(ta * s, axis=0, keepdims=True) * (1.0 / (f * hw))
    max_c = jnp.max(ta * mx, axis=0, keepdims=True)
    vc = jnp.concatenate([avg_c, max_c], axis=0)
    hc = jnp.maximum(jnp.dot(vc, wc1t_ref[...],
                             preferred_element_type=jnp.float32), 0.0)
    oc = jnp.dot(hc, wc2t_ref[...], preferred_element_type=jnp.float32)
    ca = jax.nn.sigmoid(oc[0:1, :] + oc[1:2, :])

    g_t = _wta_gate(ta)
    g_c = jnp.transpose(_wta_gate(jnp.transpose(ca)))

    g = (g_t * g_c)[:, None, None, :]
    o_ref[0] = xb * g


def kernel(x, w_ta1, w_ta2, w_ca1, w_ca2):
    b, f, c, h, w = x.shape
    xt = jnp.transpose(x, (0, 1, 3, 4, 2))
    out_t = pl.pallas_call(
        _rm_sew_body,
        grid=(b,),
        in_specs=[
            pl.BlockSpec((1, f, h, w, c), lambda i: (i, 0, 0, 0, 0)),
            pl.BlockSpec((f, f), lambda i: (0, 0)),
            pl.BlockSpec((f, f), lambda i: (0, 0)),
            pl.BlockSpec((c, c), lambda i: (0, 0)),
            pl.BlockSpec((c, c), lambda i: (0, 0)),
        ],
        out_specs=pl.BlockSpec((1, f, h, w, c), lambda i: (i, 0, 0, 0, 0)),
        out_shape=jax.ShapeDtypeStruct((b, f, h, w, c), x.dtype),
    )(xt, w_ta1, w_ta2, w_ca1.T, w_ca2.T)
    return jnp.transpose(out_t, (0, 1, 4, 2, 3))
